# Initial kernel scaffold; baseline (speedup 1.0000x reference)
import jax
import jax.numpy as jnp
from jax.experimental import pallas as pl


def kernel(support_xf, support_y, query_xf, query_y):
    raise NotImplementedError("write your pallas kernel here")


# speedup vs baseline: 4.3858x; 4.3858x over previous
import functools

import jax
import jax.numpy as jnp
from jax.experimental import pallas as pl

N_WAY = 5
K_SHOT = 5
NBNN_TOPK = 3
TEMPERATURE = 0.1
G = 32


def _mn4_kernel(qmat_ref, smat_ref, onehot_ref, out_ref, *, q, c, hw, m_s):
    ns = N_WAY * m_s
    qmat = qmat_ref[0]
    smat = smat_ref[0]
    onehot = onehot_ref[0]

    raw = jnp.dot(qmat, smat, preferred_element_type=jnp.float32)
    qn2 = jnp.sum(qmat * qmat, axis=1, keepdims=True)
    sn2 = jnp.sum(smat * smat, axis=0, keepdims=True)
    rq = 1.0 / (jnp.sqrt(qn2) + 1e-12)
    rs = 1.0 / (jnp.sqrt(sn2) + 1e-12)
    sim2 = raw * rq * rs
    sim = sim2.reshape(q, G, ns)

    iota_j = jax.lax.broadcasted_iota(jnp.int32, (1, 1, ns), 2)
    iota_i = jax.lax.broadcasted_iota(jnp.int32, (1, G, 1), 1)
    valid = iota_i < hw
    validf = valid.astype(jnp.float32)

    cwm = jnp.max(sim, axis=2, keepdims=True)
    qn = jnp.min(jnp.where(sim == cwm, iota_j, ns), axis=2, keepdims=True)

    point = (qn == iota_j) & valid
    cm = jnp.where(point, cwm + 1.0, 0.0)
    win_val = jnp.max(cm, axis=1, keepdims=True)
    win_idx = jnp.min(jnp.where(cm == win_val, iota_i, G), axis=1,
                      keepdims=True)

    mask = jnp.max((point & (win_idx == iota_i)).astype(jnp.float32),
                   axis=2, keepdims=True)
    mask = mask * validf

    iota_m = jax.lax.broadcasted_iota(jnp.int32, (1, 1, m_s), 2)
    neg = jnp.float32(-3.0e38)

    def top3_mean(chunk):
        m1 = jnp.max(chunk, axis=2, keepdims=True)
        f1 = jnp.min(jnp.where(chunk == m1, iota_m, m_s), axis=2, keepdims=True)
        chunk = jnp.where(iota_m == f1, neg, chunk)
        m2 = jnp.max(chunk, axis=2, keepdims=True)
        f2 = jnp.min(jnp.where(chunk == m2, iota_m, m_s), axis=2, keepdims=True)
        chunk = jnp.where(iota_m == f2, neg, chunk)
        m3 = jnp.max(chunk, axis=2, keepdims=True)
        return (m1 + m2 + m3) * jnp.float32(1.0 / 3.0)

    qvs = []
    for n in range(N_WAY):
        val_n = top3_mean(sim[:, :, n * m_s:(n + 1) * m_s])
        qvs.append(jnp.sum(val_n * mask, axis=1, keepdims=True))
    logits = jnp.concatenate(qvs, axis=2) * jnp.float32(1.0 / TEMPERATURE)

    lm = jnp.max(logits, axis=2, keepdims=True)
    lse = lm + jnp.log(jnp.sum(jnp.exp(logits - lm), axis=2, keepdims=True))
    logp = logits - lse
    partial = jnp.zeros((1, 1), jnp.float32) - jnp.sum(logp * onehot)

    @pl.when(pl.program_id(0) == 0)
    def _():
        out_ref[...] = jnp.zeros((1, 1), jnp.float32)

    out_ref[...] += partial


def kernel(support_xf, support_y, query_xf, query_y):
    b, q, c, h, w = query_xf.shape
    hw = h * w
    m_s = K_SHOT * hw
    ns = N_WAY * m_s

    sup = support_xf.reshape(b, N_WAY, K_SHOT, c, hw)
    sup = jnp.transpose(sup, (0, 1, 3, 2, 4)).reshape(b, N_WAY, c, m_s)
    smat = jnp.transpose(sup, (0, 2, 1, 3)).reshape(b, c, ns)

    qmat = jnp.transpose(query_xf.reshape(b, q, c, hw), (0, 1, 3, 2))
    qmat = jnp.pad(qmat, ((0, 0), (0, 0), (0, G - hw), (0, 0)))
    qmat = qmat.reshape(b, q * G, c)

    onehot = (query_y[..., None] == jnp.arange(N_WAY, dtype=query_y.dtype))
    onehot = onehot.astype(jnp.float32).reshape(b, q, 1, N_WAY)

    loss_sum = pl.pallas_call(
        functools.partial(_mn4_kernel, q=q, c=c, hw=hw, m_s=m_s),
        grid=(b,),
        in_specs=[
            pl.BlockSpec((1, q * G, c), lambda i: (i, 0, 0)),
            pl.BlockSpec((1, c, ns), lambda i: (i, 0, 0)),
            pl.BlockSpec((1, q, 1, N_WAY), lambda i: (i, 0, 0, 0)),
        ],
        out_specs=pl.BlockSpec((1, 1), lambda i: (0, 0)),
        out_shape=jax.ShapeDtypeStruct((1, 1), jnp.float32),
    )(qmat, smat, onehot)

    return loss_sum[0, 0] / (b * q)


# speedup vs baseline: 4.6114x; 1.0515x over previous
import functools

import jax
import jax.numpy as jnp
from jax.experimental import pallas as pl
from jax.experimental.pallas import tpu as pltpu

N_WAY = 5
K_SHOT = 5
NBNN_TOPK = 3
TEMPERATURE = 0.1
G = 32


def _mn4_kernel(qnat_ref, smat_ref, onehot_ref, out_ref, *, q, c, hw, m_s):
    ns = N_WAY * m_s
    f32 = jnp.float32
    qnat = qnat_ref[0]
    smat = smat_ref[0]
    onehot = onehot_ref[0]

    qpad = jnp.concatenate(
        [qnat, jnp.zeros((q, c, G - hw), f32)], axis=2)
    qmat = jnp.swapaxes(qpad, 1, 2).reshape(q * G, c)

    raw = jnp.dot(qmat, smat, preferred_element_type=f32)
    qn2 = jnp.sum(qmat * qmat, axis=1, keepdims=True)
    sn2 = jnp.sum(smat * smat, axis=0, keepdims=True)
    rq = 1.0 / (jnp.sqrt(qn2) + 1e-12)
    rs = 1.0 / (jnp.sqrt(sn2) + 1e-12)
    sim = (raw * rq * rs).reshape(q, G, ns)

    iota_j = jax.lax.broadcasted_iota(jnp.int32, (1, 1, ns), 2).astype(f32)
    iota_i = jax.lax.broadcasted_iota(jnp.int32, (1, G, 1), 1).astype(f32)
    valid = iota_i < float(hw)
    validf = valid.astype(f32)

    cwm = jnp.max(sim, axis=2, keepdims=True)
    qn = jnp.min(jnp.where(sim == cwm, iota_j, float(ns)), axis=2,
                 keepdims=True)

    point = (qn == iota_j) & valid
    cm = jnp.where(point, cwm + 1.0, 0.0)
    win_val = jnp.max(cm, axis=1, keepdims=True)
    win_idx = jnp.min(jnp.where(cm == win_val, iota_i, float(G)), axis=1,
                      keepdims=True)

    mask = jnp.max((point & (win_idx == iota_i)).astype(f32),
                   axis=2, keepdims=True)
    mask = mask * validf

    neg = f32(-3.0e38)

    def top3_sum(chunk):
        m1 = jnp.max(chunk, axis=2, keepdims=True)
        eq1 = chunk == m1
        c1 = jnp.sum(eq1.astype(f32), axis=2, keepdims=True)
        x2 = jnp.where(eq1, neg, chunk)
        m2 = jnp.max(x2, axis=2, keepdims=True)
        eq2 = x2 == m2
        c2 = jnp.sum(eq2.astype(f32), axis=2, keepdims=True)
        m3 = jnp.max(jnp.where(eq2, neg, x2), axis=2, keepdims=True)
        second = jnp.where(c1 >= 2.0, m1, m2)
        third = jnp.where(c1 >= 3.0, m1,
                          jnp.where(c1 >= 2.0, m2,
                                    jnp.where(c2 >= 2.0, m2, m3)))
        return m1 + second + third

    qvs = []
    for n in range(N_WAY):
        val_n = top3_sum(sim[:, :, n * m_s:(n + 1) * m_s])
        qvs.append(jnp.sum(val_n * mask, axis=1, keepdims=True))
    logits = jnp.concatenate(qvs, axis=2) * f32(1.0 / (3.0 * TEMPERATURE))

    lm = jnp.max(logits, axis=2, keepdims=True)
    lse = lm + jnp.log(jnp.sum(jnp.exp(logits - lm), axis=2, keepdims=True))
    logp = logits - lse
    partial = jnp.zeros((1, 1), f32) - jnp.sum(logp * onehot)

    @pl.when((pl.program_id(0) == 0) & (pl.program_id(1) == 0))
    def _():
        out_ref[...] = jnp.zeros((1, 1), f32)

    out_ref[...] += partial


def kernel(support_xf, support_y, query_xf, query_y):
    b, q, c, h, w = query_xf.shape
    hw = h * w
    m_s = K_SHOT * hw
    ns = N_WAY * m_s

    smat = jnp.transpose(support_xf.reshape(b, N_WAY * K_SHOT, c, hw),
                         (0, 2, 1, 3)).reshape(b, c, ns)

    qnat = query_xf.reshape(b, q, c, hw)

    onehot = (query_y[..., None] == jnp.arange(N_WAY, dtype=query_y.dtype))
    onehot = onehot.astype(jnp.float32).reshape(b, q, 1, N_WAY)

    qt = 15
    assert q % qt == 0
    loss_sum = pl.pallas_call(
        functools.partial(_mn4_kernel, q=qt, c=c, hw=hw, m_s=m_s),
        grid=(b, q // qt),
        in_specs=[
            pl.BlockSpec((1, qt, c, hw), lambda i, j: (i, j, 0, 0)),
            pl.BlockSpec((1, c, ns), lambda i, j: (i, 0, 0)),
            pl.BlockSpec((1, qt, 1, N_WAY), lambda i, j: (i, j, 0, 0)),
        ],
        out_specs=pl.BlockSpec((1, 1), lambda i, j: (0, 0)),
        out_shape=jax.ShapeDtypeStruct((1, 1), jnp.float32),
    )(qnat, smat, onehot)

    return loss_sum[0, 0] / (b * q)
